# Initial kernel scaffold; baseline (speedup 1.0000x reference)
import jax
import jax.numpy as jnp
from jax.experimental import pallas as pl


def kernel(pred_cls, pred_reg, anchors):
    raise NotImplementedError("write your pallas kernel here")


# speedup vs baseline: 3.5760x; 3.5760x over previous
import jax
import jax.numpy as jnp
from jax.experimental import pallas as pl
from jax.experimental.pallas import tpu as pltpu

_A = 5
_HW = 625
_N = _A * _HW
_NP = 3328
_B = 128
_PRE = 200
_POST = 30
_ROWS = 208
_IM = 255.0
_TH = 0.7
_NEG = -3.0e38


def _proposal_body(inp_ref, anc_ref, out_ref,
                   s_ref, x1_ref, y1_ref, x2_ref, y2_ref,
                   ts_ref, tx1_ref, ty1_ref, tx2_ref, ty2_ref, k_ref):
    c0 = inp_ref[0]
    c1 = inp_ref[1]
    dx = inp_ref[2]
    dy = inp_ref[3]
    dw = inp_ref[4]
    dh = inp_ref[5]

    m = jnp.maximum(c0, c1)
    score = c1 - m - jnp.log(jnp.exp(c0 - m) + jnp.exp(c1 - m))
    riota = jax.lax.broadcasted_iota(jnp.int32, (_NP, _B), 0)
    score = jnp.where(riota < _N, score, _NEG)

    cx = anc_ref[0]
    cy = anc_ref[1]
    aw = anc_ref[2]
    ah = anc_ref[3]
    pcx = dx * aw + cx
    pcy = dy * ah + cy
    pw = jnp.exp(dw) * aw
    ph = jnp.exp(dh) * ah
    x1 = jnp.clip(pcx - 0.5 * pw, 0.0, _IM - 1.0)
    y1 = jnp.clip(pcy - 0.5 * ph, 0.0, _IM - 1.0)
    x2 = jnp.clip(pcx + 0.5 * pw, 0.0, _IM - 1.0)
    y2 = jnp.clip(pcy + 0.5 * ph, 0.0, _IM - 1.0)

    s_ref[:, :] = score
    x1_ref[:, :] = x1
    y1_ref[:, :] = y1
    x2_ref[:, :] = x2
    y2_ref[:, :] = y2

    z = jnp.zeros((_ROWS, _B), jnp.float32)
    ts_ref[:, :] = z
    tx1_ref[:, :] = z
    ty1_ref[:, :] = z
    tx2_ref[:, :] = z
    ty2_ref[:, :] = z
    kio = jax.lax.broadcasted_iota(jnp.int32, (_ROWS, _B), 0)
    k_ref[:, :] = jnp.where(kio < _PRE, 1.0, 0.0)

    def sel_body(i, _):
        s = s_ref[:, :]
        mx = jnp.max(s, axis=0, keepdims=True)
        hit = s == mx
        first = jnp.min(jnp.where(hit, riota, _NP), axis=0, keepdims=True)
        oh = riota == first
        s_ref[:, :] = jnp.where(oh, _NEG, s)
        ts_ref[pl.ds(i, 1), :] = mx
        tx1_ref[pl.ds(i, 1), :] = jnp.sum(jnp.where(oh, x1_ref[:, :], 0.0), axis=0, keepdims=True)
        ty1_ref[pl.ds(i, 1), :] = jnp.sum(jnp.where(oh, y1_ref[:, :], 0.0), axis=0, keepdims=True)
        tx2_ref[pl.ds(i, 1), :] = jnp.sum(jnp.where(oh, x2_ref[:, :], 0.0), axis=0, keepdims=True)
        ty2_ref[pl.ds(i, 1), :] = jnp.sum(jnp.where(oh, y2_ref[:, :], 0.0), axis=0, keepdims=True)
        return 0

    jax.lax.fori_loop(0, _PRE, sel_body, 0)

    tx1 = tx1_ref[:, :]
    ty1 = ty1_ref[:, :]
    tx2 = tx2_ref[:, :]
    ty2 = ty2_ref[:, :]
    areas = (tx2 - tx1 + 1.0) * (ty2 - ty1 + 1.0)

    def nms_body(i, _):
        x1i = tx1_ref[pl.ds(i, 1), :]
        y1i = ty1_ref[pl.ds(i, 1), :]
        x2i = tx2_ref[pl.ds(i, 1), :]
        y2i = ty2_ref[pl.ds(i, 1), :]
        xx1 = jnp.maximum(tx1, x1i)
        yy1 = jnp.maximum(ty1, y1i)
        xx2 = jnp.minimum(tx2, x2i)
        yy2 = jnp.minimum(ty2, y2i)
        w = jnp.maximum(0.0, xx2 - xx1 + 1.0)
        h = jnp.maximum(0.0, yy2 - yy1 + 1.0)
        inter = w * h
        ai = (x2i - x1i + 1.0) * (y2i - y1i + 1.0)
        iou = inter / (areas + ai - inter)
        keep = k_ref[:, :]
        ki = k_ref[pl.ds(i, 1), :]
        supp = (iou > _TH) & (kio > i) & (ki > 0.0)
        k_ref[:, :] = jnp.where(supp, 0.0, keep)
        return 0

    jax.lax.fori_loop(0, _PRE, nms_body, 0)

    keep = k_ref[:, :]
    r = keep
    for sh in (1, 2, 4, 8, 16, 32, 64, 128):
        r = r + jnp.concatenate(
            [jnp.zeros((sh, _B), jnp.float32), r[: _ROWS - sh]], axis=0)
    rank1 = r - 1.0
    ts = ts_ref[:, :]
    kept = keep > 0.0
    for j in range(_POST):
        sel = kept & (rank1 == float(j))
        for f, arr in enumerate((ts, tx1, ty1, tx2, ty2)):
            out_ref[pl.ds(j * 5 + f, 1), :] = jnp.sum(
                jnp.where(sel, arr, 0.0), axis=0, keepdims=True)


def _run(planes, anc):
    return pl.pallas_call(
        _proposal_body,
        out_shape=jax.ShapeDtypeStruct((_POST * 5, _B), jnp.float32),
        scratch_shapes=[
            pltpu.VMEM((_NP, _B), jnp.float32),
            pltpu.VMEM((_NP, _B), jnp.float32),
            pltpu.VMEM((_NP, _B), jnp.float32),
            pltpu.VMEM((_NP, _B), jnp.float32),
            pltpu.VMEM((_NP, _B), jnp.float32),
            pltpu.VMEM((_ROWS, _B), jnp.float32),
            pltpu.VMEM((_ROWS, _B), jnp.float32),
            pltpu.VMEM((_ROWS, _B), jnp.float32),
            pltpu.VMEM((_ROWS, _B), jnp.float32),
            pltpu.VMEM((_ROWS, _B), jnp.float32),
            pltpu.VMEM((_ROWS, _B), jnp.float32),
        ],
    )(planes, anc)


@jax.jit
def kernel(pred_cls, pred_reg, anchors):
    b = pred_cls.shape[0]
    c = pred_cls.reshape(b, 2, _A, _HW)
    cls0 = c[:, 0].reshape(b, _N)
    cls1 = c[:, 1].reshape(b, _N)
    r = pred_reg.reshape(b, 4, _A, _HW).reshape(b, 4, _N)
    planes = jnp.stack([cls0, cls1, r[:, 0], r[:, 1], r[:, 2], r[:, 3]], 0)
    planes = jnp.pad(planes, ((0, 0), (0, 0), (0, _NP - _N)))
    planes = planes.transpose(0, 2, 1)
    anc = jnp.pad(anchors, ((0, _NP - _N), (0, 0))).T[:, :, None]
    out = _run(planes, anc)
    return out.reshape(_POST, 5, b).transpose(2, 0, 1)


# speedup vs baseline: 5.6613x; 1.5832x over previous
import functools
import jax
import jax.numpy as jnp
from jax import lax
from jax.experimental import pallas as pl
from jax.experimental.pallas import tpu as pltpu
from jax.experimental.pallas import tpu_sc as plsc

_A = 5
_HW = 625
_N = _A * _HW
_NP = 3456
_B = 128
_PRE = 200
_POST = 30
_CAP = 208
_CW = 5 * _CAP
_IM = 255.0
_TH = 0.7
_SENT = -2**31
_NEG = -3.0e38
_NW = 32
_IPW = _B // _NW


def _prep_body(inp_ref, anc_ref, keys_ref, vals_ref):
    c0 = inp_ref[0]
    c1 = inp_ref[1]
    dx = inp_ref[2]
    dy = inp_ref[3]
    dw = inp_ref[4]
    dh = inp_ref[5]

    m = jnp.maximum(c0, c1)
    score = c1 - m - jnp.log(jnp.exp(c0 - m) + jnp.exp(c1 - m))
    col = lax.broadcasted_iota(jnp.int32, (_B, _NP), 1)
    b = lax.bitcast_convert_type(score, jnp.int32)
    key = jnp.where(b >= 0, b, b ^ jnp.int32(0x7FFFFFFF))
    keys_ref[:, :] = jnp.where(col < _N, key, jnp.int32(_SENT))

    cx = anc_ref[0]
    cy = anc_ref[1]
    aw = anc_ref[2]
    ah = anc_ref[3]
    pcx = dx * aw + cx
    pcy = dy * ah + cy
    pw = jnp.exp(dw) * aw
    ph = jnp.exp(dh) * ah
    x1 = jnp.clip(pcx - 0.5 * pw, 0.0, _IM - 1.0)
    y1 = jnp.clip(pcy - 0.5 * ph, 0.0, _IM - 1.0)
    x2 = jnp.clip(pcx + 0.5 * pw, 0.0, _IM - 1.0)
    y2 = jnp.clip(pcy + 0.5 * ph, 0.0, _IM - 1.0)

    vals_ref[pl.ds(0 * _B, _B), :] = score
    vals_ref[pl.ds(1 * _B, _B), :] = x1
    vals_ref[pl.ds(2 * _B, _B), :] = y1
    vals_ref[pl.ds(3 * _B, _B), :] = x2
    vals_ref[pl.ds(4 * _B, _B), :] = y2


def _prep(planes, anc):
    return pl.pallas_call(
        _prep_body,
        out_shape=[
            jax.ShapeDtypeStruct((_B, _NP), jnp.int32),
            jax.ShapeDtypeStruct((5 * _B, _NP), jnp.float32),
        ],
    )(planes, anc)


def _sc_body(keys_hbm, vals_hbm, cand_hbm,
             key_v, val0_v, val1_v, val2_v, val3_v, val4_v, out_v):
    wid = lax.axis_index("s") * 2 + lax.axis_index("c")
    iota16 = lax.iota(jnp.int32, 16)
    zi = jnp.zeros((16,), jnp.int32)
    zf = jnp.zeros((16,), jnp.float32)

    def bsum(v):
        for d in (8, 4, 2, 1):
            v = v + v.at[iota16 ^ d].get(mode="promise_in_bounds")
        return v

    def excl_prefix(v):
        x = v
        for d in (1, 2, 4, 8):
            sh = x.at[jnp.maximum(iota16 - d, 0)].get(mode="promise_in_bounds")
            x = x + jnp.where(iota16 >= d, sh, 0)
        return x - v

    def one_image(n, _):
        img = wid * _IPW + n

        pltpu.sync_copy(keys_hbm.at[pl.ds(img * _NP, _NP)], key_v)
        for f, vref in enumerate((val0_v, val1_v, val2_v, val3_v, val4_v)):
            pltpu.sync_copy(
                vals_hbm.at[pl.ds((f * _B + img) * _NP, _NP)], vref)

        ut = jnp.uint32(0)
        for bit in range(31, -1, -1):
            ut2 = ut | jnp.uint32(1 << bit)
            st2 = (ut2 ^ jnp.uint32(0x80000000)).astype(jnp.int32)

            def cbody(r, acc, st2=st2):
                for j in range(8):
                    kv = key_v[pl.ds(r * 128 + j * 16, 16)]
                    acc = acc + jnp.where(kv >= st2, 1, 0).astype(jnp.int32)
                return acc

            acc = lax.fori_loop(0, _NP // 128, cbody, zi)
            cnt = bsum(acc)[0]
            ut = jnp.where(cnt >= _PRE, ut2, ut)
        tau = (ut ^ jnp.uint32(0x80000000)).astype(jnp.int32)

        def gbody(r, acc):
            for j in range(8):
                kv = key_v[pl.ds(r * 128 + j * 16, 16)]
                acc = acc + jnp.where(kv > tau, 1, 0).astype(jnp.int32)
            return acc

        c1n = bsum(lax.fori_loop(0, _NP // 128, gbody, zi))[0]
        needed = _PRE - c1n

        th = jnp.int32(0)
        for bit in range(11, -1, -1):
            cand = th + jnp.int32((1 << bit) - 1)

            def tbody(r, acc, cand=cand):
                for j in range(8):
                    off = r * 128 + j * 16
                    kv = key_v[pl.ds(off, 16)]
                    msk = (kv == tau) & (iota16 + off <= cand)
                    acc = acc + jnp.where(msk, 1, 0).astype(jnp.int32)
                return acc

            ec = bsum(lax.fori_loop(0, _NP // 128, tbody, zi))[0]
            th = jnp.where(ec >= needed, th, th + jnp.int32(1 << bit))

        for r in range(_CAP // 16):
            out_v[pl.ds(r * 16, 16)] = jnp.full((16,), _NEG, jnp.float32)
        for r in range(_CAP // 16, _CW // 16):
            out_v[pl.ds(r * 16, 16)] = zf

        def kbody(r, base):
            off = r * 16
            kv = key_v[pl.ds(off, 16)]
            msel = (kv > tau) | ((kv == tau) & (iota16 + off <= th))
            mi = jnp.where(msel, 1, 0).astype(jnp.int32)
            pfx = excl_prefix(mi)
            dest = base + pfx
            for f, vref in enumerate((val0_v, val1_v, val2_v, val3_v, val4_v)):
                plsc.store_scatter(out_v, [dest + f * _CAP],
                                   vref[pl.ds(off, 16)], mask=msel)
            return base + pfx[15] + mi[15]

        lax.fori_loop(0, _NP // 16, kbody, jnp.int32(0))

        pltpu.sync_copy(out_v, cand_hbm.at[pl.ds(img * _CW, _CW)])
        return 0

    lax.fori_loop(0, _IPW, one_image, 0)


_sc_call = functools.partial(
    pl.kernel,
    out_type=jax.ShapeDtypeStruct((_B * _CW,), jnp.float32),
    mesh=plsc.VectorSubcoreMesh(core_axis_name="c", subcore_axis_name="s"),
    compiler_params=pltpu.CompilerParams(needs_layout_passes=False),
    scratch_types=[
        pltpu.VMEM((_NP,), jnp.int32),
        pltpu.VMEM((_NP,), jnp.float32),
        pltpu.VMEM((_NP,), jnp.float32),
        pltpu.VMEM((_NP,), jnp.float32),
        pltpu.VMEM((_NP,), jnp.float32),
        pltpu.VMEM((_NP,), jnp.float32),
        pltpu.VMEM((_CW,), jnp.float32),
    ],
)(_sc_body)


def _nms_body(cand_ref, out_ref, ts_ref, tx1_ref, ty1_ref, tx2_ref, ty2_ref,
              k_ref):
    sc = jnp.transpose(cand_ref[:, pl.ds(0 * _CAP, _CAP)])
    bx1 = jnp.transpose(cand_ref[:, pl.ds(1 * _CAP, _CAP)])
    by1 = jnp.transpose(cand_ref[:, pl.ds(2 * _CAP, _CAP)])
    bx2 = jnp.transpose(cand_ref[:, pl.ds(3 * _CAP, _CAP)])
    by2 = jnp.transpose(cand_ref[:, pl.ds(4 * _CAP, _CAP)])

    riota = lax.broadcasted_iota(jnp.int32, (_CAP, _B), 0)

    def sel_pre(i, carry):
        s, x1, y1, x2, y2 = carry
        mx = jnp.max(s, axis=0, keepdims=True)
        first = jnp.min(jnp.where(s == mx, riota, _CAP),
                        axis=0, keepdims=True)
        oh = riota == first
        ts_ref[pl.ds(i, 1), :] = mx
        tx1_ref[pl.ds(i, 1), :] = jnp.sum(jnp.where(oh, x1, 0.0),
                                          axis=0, keepdims=True)
        ty1_ref[pl.ds(i, 1), :] = jnp.sum(jnp.where(oh, y1, 0.0),
                                          axis=0, keepdims=True)
        tx2_ref[pl.ds(i, 1), :] = jnp.sum(jnp.where(oh, x2, 0.0),
                                          axis=0, keepdims=True)
        ty2_ref[pl.ds(i, 1), :] = jnp.sum(jnp.where(oh, y2, 0.0),
                                          axis=0, keepdims=True)
        return (jnp.where(oh, _NEG, s), x1, y1, x2, y2)

    lax.fori_loop(0, _PRE, sel_pre, (sc, bx1, by1, bx2, by2))

    kio = riota
    k_ref[:, :] = jnp.where(kio < _PRE, 1.0, 0.0)
    tx1 = tx1_ref[:, :]
    ty1 = ty1_ref[:, :]
    tx2 = tx2_ref[:, :]
    ty2 = ty2_ref[:, :]
    areas = (tx2 - tx1 + 1.0) * (ty2 - ty1 + 1.0)

    def nms_step(i, _):
        x1i = tx1_ref[pl.ds(i, 1), :]
        y1i = ty1_ref[pl.ds(i, 1), :]
        x2i = tx2_ref[pl.ds(i, 1), :]
        y2i = ty2_ref[pl.ds(i, 1), :]
        xx1 = jnp.maximum(tx1, x1i)
        yy1 = jnp.maximum(ty1, y1i)
        xx2 = jnp.minimum(tx2, x2i)
        yy2 = jnp.minimum(ty2, y2i)
        w = jnp.maximum(0.0, xx2 - xx1 + 1.0)
        h = jnp.maximum(0.0, yy2 - yy1 + 1.0)
        inter = w * h
        ai = (x2i - x1i + 1.0) * (y2i - y1i + 1.0)
        iou = inter / (areas + ai - inter)
        keep = k_ref[:, :]
        ki = k_ref[pl.ds(i, 1), :]
        supp = (iou > _TH) & (kio > i) & (ki > 0.0)
        k_ref[:, :] = jnp.where(supp, 0.0, keep)
        return 0

    lax.fori_loop(0, _PRE, nms_step, 0)

    keep = k_ref[:, :]
    r = keep
    for sh in (1, 2, 4, 8, 16, 32, 64, 128):
        r = r + jnp.concatenate(
            [jnp.zeros((sh, _B), jnp.float32), r[: _CAP - sh]], axis=0)
    rank1 = r - 1.0
    ts = ts_ref[:, :]
    kept = keep > 0.0
    for j in range(_POST):
        sel = kept & (rank1 == float(j))
        for f, arr in enumerate((ts, tx1, ty1, tx2, ty2)):
            out_ref[pl.ds(j * 5 + f, 1), :] = jnp.sum(
                jnp.where(sel, arr, 0.0), axis=0, keepdims=True)


def _nms(cand):
    return pl.pallas_call(
        _nms_body,
        out_shape=jax.ShapeDtypeStruct((_POST * 5, _B), jnp.float32),
        scratch_shapes=[
            pltpu.VMEM((_CAP, _B), jnp.float32),
            pltpu.VMEM((_CAP, _B), jnp.float32),
            pltpu.VMEM((_CAP, _B), jnp.float32),
            pltpu.VMEM((_CAP, _B), jnp.float32),
            pltpu.VMEM((_CAP, _B), jnp.float32),
            pltpu.VMEM((_CAP, _B), jnp.float32),
        ],
    )(cand)


@jax.jit
def kernel(pred_cls, pred_reg, anchors):
    b = pred_cls.shape[0]
    c = pred_cls.reshape(b, 2, _A, _HW)
    cls0 = c[:, 0].reshape(b, _N)
    cls1 = c[:, 1].reshape(b, _N)
    r = pred_reg.reshape(b, 4, _A, _HW).reshape(b, 4, _N)
    planes = jnp.stack([cls0, cls1, r[:, 0], r[:, 1], r[:, 2], r[:, 3]], 0)
    planes = jnp.pad(planes, ((0, 0), (0, 0), (0, _NP - _N)))
    anc = jnp.pad(anchors, ((0, _NP - _N), (0, 0))).T[:, None, :]
    keys, vals = _prep(planes, anc)
    cand = _sc_call(keys.reshape(-1), vals.reshape(-1))
    out = _nms(cand.reshape(_B, _CW))
    return out.reshape(_POST, 5, b).transpose(2, 0, 1)


# speedup vs baseline: 6.0220x; 1.0637x over previous
import functools
import jax
import jax.numpy as jnp
from jax import lax
from jax.experimental import pallas as pl
from jax.experimental.pallas import tpu as pltpu
from jax.experimental.pallas import tpu_sc as plsc

_A = 5
_HW = 625
_N = _A * _HW
_NP = 3456
_B = 128
_PRE = 200
_POST = 30
_CAP = 208
_CW = 5 * _CAP
_IM = 255.0
_TH = 0.7
_SENT = -2**31
_NEG = -3.0e38
_NW = 32
_IPW = _B // _NW


def _prep_body(inp_ref, anc_ref, keys_ref, vals_ref):
    c0 = inp_ref[0]
    c1 = inp_ref[1]
    dx = inp_ref[2]
    dy = inp_ref[3]
    dw = inp_ref[4]
    dh = inp_ref[5]

    m = jnp.maximum(c0, c1)
    score = c1 - m - jnp.log(jnp.exp(c0 - m) + jnp.exp(c1 - m))
    col = lax.broadcasted_iota(jnp.int32, (_B, _NP), 1)
    b = lax.bitcast_convert_type(score, jnp.int32)
    key = jnp.where(b >= 0, b, b ^ jnp.int32(0x7FFFFFFF))
    keys_ref[:, :] = jnp.where(col < _N, key, jnp.int32(_SENT))

    cx = anc_ref[0]
    cy = anc_ref[1]
    aw = anc_ref[2]
    ah = anc_ref[3]
    pcx = dx * aw + cx
    pcy = dy * ah + cy
    pw = jnp.exp(dw) * aw
    ph = jnp.exp(dh) * ah
    x1 = jnp.clip(pcx - 0.5 * pw, 0.0, _IM - 1.0)
    y1 = jnp.clip(pcy - 0.5 * ph, 0.0, _IM - 1.0)
    x2 = jnp.clip(pcx + 0.5 * pw, 0.0, _IM - 1.0)
    y2 = jnp.clip(pcy + 0.5 * ph, 0.0, _IM - 1.0)

    vals_ref[pl.ds(0 * _B, _B), :] = score
    vals_ref[pl.ds(1 * _B, _B), :] = x1
    vals_ref[pl.ds(2 * _B, _B), :] = y1
    vals_ref[pl.ds(3 * _B, _B), :] = x2
    vals_ref[pl.ds(4 * _B, _B), :] = y2


def _prep(planes, anc):
    return pl.pallas_call(
        _prep_body,
        out_shape=[
            jax.ShapeDtypeStruct((_B, _NP), jnp.int32),
            jax.ShapeDtypeStruct((5 * _B, _NP), jnp.float32),
        ],
    )(planes, anc)


def _sc_body(keys_hbm, vals_hbm, cand_hbm,
             key_v, val0_v, val1_v, val2_v, val3_v, val4_v, out_v):
    wid = lax.axis_index("s") * 2 + lax.axis_index("c")
    iota16 = lax.iota(jnp.int32, 16)
    zi = jnp.zeros((16,), jnp.int32)
    zf = jnp.zeros((16,), jnp.float32)

    def bsum(v):
        for d in (8, 4, 2, 1):
            v = v + v.at[iota16 ^ d].get(mode="promise_in_bounds")
        return v

    def excl_prefix(v):
        x = v
        for d in (1, 2, 4, 8):
            sh = x.at[jnp.maximum(iota16 - d, 0)].get(mode="promise_in_bounds")
            x = x + jnp.where(iota16 >= d, sh, 0)
        return x - v

    def one_image(n, _):
        img = wid * _IPW + n

        pltpu.sync_copy(keys_hbm.at[pl.ds(img * _NP, _NP)], key_v)
        for f, vref in enumerate((val0_v, val1_v, val2_v, val3_v, val4_v)):
            pltpu.sync_copy(
                vals_hbm.at[pl.ds((f * _B + img) * _NP, _NP)], vref)

        ut = jnp.uint32(0)
        cge = jnp.int32(_N)
        for bit in range(31, -1, -1):
            ut2 = ut | jnp.uint32(1 << bit)
            st2 = (ut2 ^ jnp.uint32(0x80000000)).astype(jnp.int32)

            def cbody(r, acc, st2=st2):
                for j in range(8):
                    kv = key_v[pl.ds(r * 128 + j * 16, 16)]
                    acc = acc + jnp.where(kv >= st2, 1, 0).astype(jnp.int32)
                return acc

            acc = lax.fori_loop(0, _NP // 128, cbody, zi)
            cnt = bsum(acc)[0]
            take = cnt >= _PRE
            ut = jnp.where(take, ut2, ut)
            cge = jnp.where(take, cnt, cge)
        tau = (ut ^ jnp.uint32(0x80000000)).astype(jnp.int32)

        def tie_search(_):
            def gbody(r, acc):
                for j in range(8):
                    kv = key_v[pl.ds(r * 128 + j * 16, 16)]
                    acc = acc + jnp.where(kv > tau, 1, 0).astype(jnp.int32)
                return acc

            c1n = bsum(lax.fori_loop(0, _NP // 128, gbody, zi))[0]
            needed = _PRE - c1n

            th = jnp.int32(0)
            for bit in range(11, -1, -1):
                cand = th + jnp.int32((1 << bit) - 1)

                def tbody(r, acc, cand=cand):
                    for j in range(8):
                        off = r * 128 + j * 16
                        kv = key_v[pl.ds(off, 16)]
                        msk = (kv == tau) & (iota16 + off <= cand)
                        acc = acc + jnp.where(msk, 1, 0).astype(jnp.int32)
                    return acc

                ec = bsum(lax.fori_loop(0, _NP // 128, tbody, zi))[0]
                th = jnp.where(ec >= needed, th, th + jnp.int32(1 << bit))
            return th

        th = lax.cond(cge > _PRE, tie_search,
                      lambda _: jnp.int32(_NP), 0)

        for r in range(_CAP // 16):
            out_v[pl.ds(r * 16, 16)] = jnp.full((16,), _NEG, jnp.float32)
        for r in range(_CAP // 16, _CW // 16):
            out_v[pl.ds(r * 16, 16)] = zf

        def kbody(r, base):
            off = r * 16
            kv = key_v[pl.ds(off, 16)]
            msel = (kv > tau) | ((kv == tau) & (iota16 + off <= th))
            mi = jnp.where(msel, 1, 0).astype(jnp.int32)
            pfx = excl_prefix(mi)
            dest = base + pfx
            for f, vref in enumerate((val0_v, val1_v, val2_v, val3_v, val4_v)):
                plsc.store_scatter(out_v, [dest + f * _CAP],
                                   vref[pl.ds(off, 16)], mask=msel)
            return base + pfx[15] + mi[15]

        lax.fori_loop(0, _NP // 16, kbody, jnp.int32(0))

        pltpu.sync_copy(out_v, cand_hbm.at[pl.ds(img * _CW, _CW)])
        return 0

    lax.fori_loop(0, _IPW, one_image, 0)


_sc_call = functools.partial(
    pl.kernel,
    out_type=jax.ShapeDtypeStruct((_B * _CW,), jnp.float32),
    mesh=plsc.VectorSubcoreMesh(core_axis_name="c", subcore_axis_name="s"),
    compiler_params=pltpu.CompilerParams(needs_layout_passes=False),
    scratch_types=[
        pltpu.VMEM((_NP,), jnp.int32),
        pltpu.VMEM((_NP,), jnp.float32),
        pltpu.VMEM((_NP,), jnp.float32),
        pltpu.VMEM((_NP,), jnp.float32),
        pltpu.VMEM((_NP,), jnp.float32),
        pltpu.VMEM((_NP,), jnp.float32),
        pltpu.VMEM((_CW,), jnp.float32),
    ],
)(_sc_body)


def _nms_body(cand_ref, out_ref, ts_ref, tx1_ref, ty1_ref, tx2_ref, ty2_ref,
              k_ref):
    sc = jnp.transpose(cand_ref[:, pl.ds(0 * _CAP, _CAP)])
    bx1 = jnp.transpose(cand_ref[:, pl.ds(1 * _CAP, _CAP)])
    by1 = jnp.transpose(cand_ref[:, pl.ds(2 * _CAP, _CAP)])
    bx2 = jnp.transpose(cand_ref[:, pl.ds(3 * _CAP, _CAP)])
    by2 = jnp.transpose(cand_ref[:, pl.ds(4 * _CAP, _CAP)])

    riota = lax.broadcasted_iota(jnp.int32, (_CAP, _B), 0)

    def sel_pre(i, carry):
        s, x1, y1, x2, y2 = carry
        mx = jnp.max(s, axis=0, keepdims=True)
        first = jnp.min(jnp.where(s == mx, riota, _CAP),
                        axis=0, keepdims=True)
        oh = riota == first
        ts_ref[pl.ds(i, 1), :] = mx
        tx1_ref[pl.ds(i, 1), :] = jnp.sum(jnp.where(oh, x1, 0.0),
                                          axis=0, keepdims=True)
        ty1_ref[pl.ds(i, 1), :] = jnp.sum(jnp.where(oh, y1, 0.0),
                                          axis=0, keepdims=True)
        tx2_ref[pl.ds(i, 1), :] = jnp.sum(jnp.where(oh, x2, 0.0),
                                          axis=0, keepdims=True)
        ty2_ref[pl.ds(i, 1), :] = jnp.sum(jnp.where(oh, y2, 0.0),
                                          axis=0, keepdims=True)
        return (jnp.where(oh, _NEG, s), x1, y1, x2, y2)

    lax.fori_loop(0, _PRE, sel_pre, (sc, bx1, by1, bx2, by2))

    kio = riota
    k_ref[:, :] = jnp.where(kio < _PRE, 1.0, 0.0)
    tx1 = tx1_ref[:, :]
    ty1 = ty1_ref[:, :]
    tx2 = tx2_ref[:, :]
    ty2 = ty2_ref[:, :]
    areas = (tx2 - tx1 + 1.0) * (ty2 - ty1 + 1.0)

    def nms_step(i, _):
        x1i = tx1_ref[pl.ds(i, 1), :]
        y1i = ty1_ref[pl.ds(i, 1), :]
        x2i = tx2_ref[pl.ds(i, 1), :]
        y2i = ty2_ref[pl.ds(i, 1), :]
        xx1 = jnp.maximum(tx1, x1i)
        yy1 = jnp.maximum(ty1, y1i)
        xx2 = jnp.minimum(tx2, x2i)
        yy2 = jnp.minimum(ty2, y2i)
        w = jnp.maximum(0.0, xx2 - xx1 + 1.0)
        h = jnp.maximum(0.0, yy2 - yy1 + 1.0)
        inter = w * h
        ai = (x2i - x1i + 1.0) * (y2i - y1i + 1.0)
        iou = inter / (areas + ai - inter)
        keep = k_ref[:, :]
        ki = k_ref[pl.ds(i, 1), :]
        supp = (iou > _TH) & (kio > i) & (ki > 0.0)
        k_ref[:, :] = jnp.where(supp, 0.0, keep)
        return 0

    lax.fori_loop(0, _PRE, nms_step, 0)

    keep = k_ref[:, :]
    r = keep
    for sh in (1, 2, 4, 8, 16, 32, 64, 128):
        r = r + jnp.concatenate(
            [jnp.zeros((sh, _B), jnp.float32), r[: _CAP - sh]], axis=0)
    rank1 = r - 1.0
    ts = ts_ref[:, :]
    kept = keep > 0.0
    for j in range(_POST):
        sel = kept & (rank1 == float(j))
        for f, arr in enumerate((ts, tx1, ty1, tx2, ty2)):
            out_ref[pl.ds(j * 5 + f, 1), :] = jnp.sum(
                jnp.where(sel, arr, 0.0), axis=0, keepdims=True)


def _nms(cand):
    return pl.pallas_call(
        _nms_body,
        out_shape=jax.ShapeDtypeStruct((_POST * 5, _B), jnp.float32),
        scratch_shapes=[
            pltpu.VMEM((_CAP, _B), jnp.float32),
            pltpu.VMEM((_CAP, _B), jnp.float32),
            pltpu.VMEM((_CAP, _B), jnp.float32),
            pltpu.VMEM((_CAP, _B), jnp.float32),
            pltpu.VMEM((_CAP, _B), jnp.float32),
            pltpu.VMEM((_CAP, _B), jnp.float32),
        ],
    )(cand)


@jax.jit
def kernel(pred_cls, pred_reg, anchors):
    b = pred_cls.shape[0]
    c = pred_cls.reshape(b, 2, _A, _HW)
    cls0 = c[:, 0].reshape(b, _N)
    cls1 = c[:, 1].reshape(b, _N)
    r = pred_reg.reshape(b, 4, _A, _HW).reshape(b, 4, _N)
    planes = jnp.stack([cls0, cls1, r[:, 0], r[:, 1], r[:, 2], r[:, 3]], 0)
    planes = jnp.pad(planes, ((0, 0), (0, 0), (0, _NP - _N)))
    anc = jnp.pad(anchors, ((0, _NP - _N), (0, 0))).T[:, None, :]
    keys, vals = _prep(planes, anc)
    cand = _sc_call(keys.reshape(-1), vals.reshape(-1))
    out = _nms(cand.reshape(_B, _CW))
    return out.reshape(_POST, 5, b).transpose(2, 0, 1)


# speedup vs baseline: 6.3358x; 1.0521x over previous
import functools
import jax
import jax.numpy as jnp
from jax import lax
from jax.experimental import pallas as pl
from jax.experimental.pallas import tpu as pltpu
from jax.experimental.pallas import tpu_sc as plsc

_A = 5
_HW = 625
_N = _A * _HW
_NP = 3456
_B = 128
_PRE = 200
_POST = 30
_CAP = 208
_CW = 5 * _CAP
_IM = 255.0
_TH = 0.7
_SENT = -2**31
_NEG = -3.0e38
_NW = 32
_IPW = _B // _NW


def _prep_body(inp_ref, anc_ref, keys_ref, vals_ref):
    c0 = inp_ref[0]
    c1 = inp_ref[1]
    dx = inp_ref[2]
    dy = inp_ref[3]
    dw = inp_ref[4]
    dh = inp_ref[5]

    m = jnp.maximum(c0, c1)
    score = c1 - m - jnp.log(jnp.exp(c0 - m) + jnp.exp(c1 - m))
    col = lax.broadcasted_iota(jnp.int32, (_B, _NP), 1)
    b = lax.bitcast_convert_type(score, jnp.int32)
    key = jnp.where(b >= 0, b, b ^ jnp.int32(0x7FFFFFFF))
    keys_ref[:, :] = jnp.where(col < _N, key, jnp.int32(_SENT))

    cx = anc_ref[0]
    cy = anc_ref[1]
    aw = anc_ref[2]
    ah = anc_ref[3]
    pcx = dx * aw + cx
    pcy = dy * ah + cy
    pw = jnp.exp(dw) * aw
    ph = jnp.exp(dh) * ah
    x1 = jnp.clip(pcx - 0.5 * pw, 0.0, _IM - 1.0)
    y1 = jnp.clip(pcy - 0.5 * ph, 0.0, _IM - 1.0)
    x2 = jnp.clip(pcx + 0.5 * pw, 0.0, _IM - 1.0)
    y2 = jnp.clip(pcy + 0.5 * ph, 0.0, _IM - 1.0)

    vals_ref[pl.ds(0 * _B, _B), :] = score
    vals_ref[pl.ds(1 * _B, _B), :] = x1
    vals_ref[pl.ds(2 * _B, _B), :] = y1
    vals_ref[pl.ds(3 * _B, _B), :] = x2
    vals_ref[pl.ds(4 * _B, _B), :] = y2


def _prep(planes, anc):
    return pl.pallas_call(
        _prep_body,
        out_shape=[
            jax.ShapeDtypeStruct((_B, _NP), jnp.int32),
            jax.ShapeDtypeStruct((5 * _B, _NP), jnp.float32),
        ],
    )(planes, anc)


def _sc_body(keys_hbm, vals_hbm, cand_hbm,
             key_v, val0_v, val1_v, val2_v, val3_v, val4_v, out_v):
    wid = lax.axis_index("s") * 2 + lax.axis_index("c")
    iota16 = lax.iota(jnp.int32, 16)
    zi = jnp.zeros((16,), jnp.int32)
    zf = jnp.zeros((16,), jnp.float32)

    def bsum(v):
        for d in (8, 4, 2, 1):
            v = v + v.at[iota16 ^ d].get(mode="promise_in_bounds")
        return v

    def excl_prefix(v):
        x = v
        for d in (1, 2, 4, 8):
            sh = x.at[jnp.maximum(iota16 - d, 0)].get(mode="promise_in_bounds")
            x = x + jnp.where(iota16 >= d, sh, 0)
        return x - v

    img0 = wid * _IPW
    pltpu.sync_copy(keys_hbm.at[pl.ds(img0 * _NP, _IPW * _NP)], key_v)
    for f, vref in enumerate((val0_v, val1_v, val2_v, val3_v, val4_v)):
        pltpu.sync_copy(
            vals_hbm.at[pl.ds((f * _B + img0) * _NP, _IPW * _NP)], vref)

    def one_image(n, _):
        nb = n * _NP

        ut = jnp.uint32(0)
        cge = jnp.int32(_N)
        for bit in range(31, -1, -1):
            ut2 = ut | jnp.uint32(1 << bit)
            st2 = (ut2 ^ jnp.uint32(0x80000000)).astype(jnp.int32)

            def cbody(r, acc, st2=st2):
                for j in range(8):
                    kv = key_v[pl.ds(nb + r * 128 + j * 16, 16)]
                    acc = acc + jnp.where(kv >= st2, 1, 0).astype(jnp.int32)
                return acc

            acc = lax.fori_loop(0, _NP // 128, cbody, zi)
            cnt = bsum(acc)[0]
            take = cnt >= _PRE
            ut = jnp.where(take, ut2, ut)
            cge = jnp.where(take, cnt, cge)
        tau = (ut ^ jnp.uint32(0x80000000)).astype(jnp.int32)

        def tie_search(_):
            def gbody(r, acc):
                for j in range(8):
                    kv = key_v[pl.ds(nb + r * 128 + j * 16, 16)]
                    acc = acc + jnp.where(kv > tau, 1, 0).astype(jnp.int32)
                return acc

            c1n = bsum(lax.fori_loop(0, _NP // 128, gbody, zi))[0]
            needed = _PRE - c1n

            th = jnp.int32(0)
            for bit in range(11, -1, -1):
                cand = th + jnp.int32((1 << bit) - 1)

                def tbody(r, acc, cand=cand):
                    for j in range(8):
                        off = r * 128 + j * 16
                        kv = key_v[pl.ds(nb + off, 16)]
                        msk = (kv == tau) & (iota16 + off <= cand)
                        acc = acc + jnp.where(msk, 1, 0).astype(jnp.int32)
                    return acc

                ec = bsum(lax.fori_loop(0, _NP // 128, tbody, zi))[0]
                th = jnp.where(ec >= needed, th, th + jnp.int32(1 << bit))
            return th

        th = lax.cond(cge > _PRE, tie_search,
                      lambda _: jnp.int32(_NP), 0)

        for r in range(_CAP // 16):
            out_v[pl.ds(n * _CW + r * 16, 16)] = jnp.full((16,), _NEG,
                                                          jnp.float32)
        for r in range(_CAP // 16, _CW // 16):
            out_v[pl.ds(n * _CW + r * 16, 16)] = zf

        def kbody(r, base):
            off = r * 16
            kv = key_v[pl.ds(nb + off, 16)]
            msel = (kv > tau) | ((kv == tau) & (iota16 + off <= th))
            mi = jnp.where(msel, 1, 0).astype(jnp.int32)
            pfx = excl_prefix(mi)
            dest = base + pfx
            for f, vref in enumerate((val0_v, val1_v, val2_v, val3_v, val4_v)):
                plsc.store_scatter(out_v, [n * _CW + dest + f * _CAP],
                                   vref[pl.ds(nb + off, 16)], mask=msel)
            return base + pfx[15] + mi[15]

        lax.fori_loop(0, _NP // 16, kbody, jnp.int32(0))

        return 0

    lax.fori_loop(0, _IPW, one_image, 0)
    pltpu.sync_copy(out_v, cand_hbm.at[pl.ds(img0 * _CW, _IPW * _CW)])


_sc_call = functools.partial(
    pl.kernel,
    out_type=jax.ShapeDtypeStruct((_B * _CW,), jnp.float32),
    mesh=plsc.VectorSubcoreMesh(core_axis_name="c", subcore_axis_name="s"),
    compiler_params=pltpu.CompilerParams(needs_layout_passes=False),
    scratch_types=[
        pltpu.VMEM((_IPW * _NP,), jnp.int32),
        pltpu.VMEM((_IPW * _NP,), jnp.float32),
        pltpu.VMEM((_IPW * _NP,), jnp.float32),
        pltpu.VMEM((_IPW * _NP,), jnp.float32),
        pltpu.VMEM((_IPW * _NP,), jnp.float32),
        pltpu.VMEM((_IPW * _NP,), jnp.float32),
        pltpu.VMEM((_IPW * _CW,), jnp.float32),
    ],
)(_sc_body)


def _nms_body(cand_ref, out_ref, ts_ref, tx1_ref, ty1_ref, tx2_ref, ty2_ref,
              k_ref):
    sc = jnp.transpose(cand_ref[:, pl.ds(0 * _CAP, _CAP)])
    bx1 = jnp.transpose(cand_ref[:, pl.ds(1 * _CAP, _CAP)])
    by1 = jnp.transpose(cand_ref[:, pl.ds(2 * _CAP, _CAP)])
    bx2 = jnp.transpose(cand_ref[:, pl.ds(3 * _CAP, _CAP)])
    by2 = jnp.transpose(cand_ref[:, pl.ds(4 * _CAP, _CAP)])

    riota = lax.broadcasted_iota(jnp.int32, (_CAP, _B), 0)

    def sel_pre(i, carry):
        s, x1, y1, x2, y2 = carry
        mx = jnp.max(s, axis=0, keepdims=True)
        first = jnp.min(jnp.where(s == mx, riota, _CAP),
                        axis=0, keepdims=True)
        oh = riota == first
        ts_ref[pl.ds(i, 1), :] = mx
        tx1_ref[pl.ds(i, 1), :] = jnp.sum(jnp.where(oh, x1, 0.0),
                                          axis=0, keepdims=True)
        ty1_ref[pl.ds(i, 1), :] = jnp.sum(jnp.where(oh, y1, 0.0),
                                          axis=0, keepdims=True)
        tx2_ref[pl.ds(i, 1), :] = jnp.sum(jnp.where(oh, x2, 0.0),
                                          axis=0, keepdims=True)
        ty2_ref[pl.ds(i, 1), :] = jnp.sum(jnp.where(oh, y2, 0.0),
                                          axis=0, keepdims=True)
        return (jnp.where(oh, _NEG, s), x1, y1, x2, y2)

    lax.fori_loop(0, _PRE, sel_pre, (sc, bx1, by1, bx2, by2))

    kio = riota
    k_ref[:, :] = jnp.where(kio < _PRE, 1.0, 0.0)
    tx1 = tx1_ref[:, :]
    ty1 = ty1_ref[:, :]
    tx2 = tx2_ref[:, :]
    ty2 = ty2_ref[:, :]
    areas = (tx2 - tx1 + 1.0) * (ty2 - ty1 + 1.0)

    def nms_step(i, _):
        x1i = tx1_ref[pl.ds(i, 1), :]
        y1i = ty1_ref[pl.ds(i, 1), :]
        x2i = tx2_ref[pl.ds(i, 1), :]
        y2i = ty2_ref[pl.ds(i, 1), :]
        xx1 = jnp.maximum(tx1, x1i)
        yy1 = jnp.maximum(ty1, y1i)
        xx2 = jnp.minimum(tx2, x2i)
        yy2 = jnp.minimum(ty2, y2i)
        w = jnp.maximum(0.0, xx2 - xx1 + 1.0)
        h = jnp.maximum(0.0, yy2 - yy1 + 1.0)
        inter = w * h
        ai = (x2i - x1i + 1.0) * (y2i - y1i + 1.0)
        iou = inter / (areas + ai - inter)
        keep = k_ref[:, :]
        ki = k_ref[pl.ds(i, 1), :]
        supp = (iou > _TH) & (kio > i) & (ki > 0.0)
        k_ref[:, :] = jnp.where(supp, 0.0, keep)
        return 0

    lax.fori_loop(0, _PRE, nms_step, 0)

    keep = k_ref[:, :]
    r = keep
    for sh in (1, 2, 4, 8, 16, 32, 64, 128):
        r = r + jnp.concatenate(
            [jnp.zeros((sh, _B), jnp.float32), r[: _CAP - sh]], axis=0)
    rank1 = r - 1.0
    ts = ts_ref[:, :]
    kept = keep > 0.0
    for j in range(_POST):
        sel = kept & (rank1 == float(j))
        for f, arr in enumerate((ts, tx1, ty1, tx2, ty2)):
            out_ref[pl.ds(j * 5 + f, 1), :] = jnp.sum(
                jnp.where(sel, arr, 0.0), axis=0, keepdims=True)


def _nms(cand):
    return pl.pallas_call(
        _nms_body,
        out_shape=jax.ShapeDtypeStruct((_POST * 5, _B), jnp.float32),
        scratch_shapes=[
            pltpu.VMEM((_CAP, _B), jnp.float32),
            pltpu.VMEM((_CAP, _B), jnp.float32),
            pltpu.VMEM((_CAP, _B), jnp.float32),
            pltpu.VMEM((_CAP, _B), jnp.float32),
            pltpu.VMEM((_CAP, _B), jnp.float32),
            pltpu.VMEM((_CAP, _B), jnp.float32),
        ],
    )(cand)


@jax.jit
def kernel(pred_cls, pred_reg, anchors):
    b = pred_cls.shape[0]
    c = pred_cls.reshape(b, 2, _A, _HW)
    cls0 = c[:, 0].reshape(b, _N)
    cls1 = c[:, 1].reshape(b, _N)
    r = pred_reg.reshape(b, 4, _A, _HW).reshape(b, 4, _N)
    planes = jnp.stack([cls0, cls1, r[:, 0], r[:, 1], r[:, 2], r[:, 3]], 0)
    planes = jnp.pad(planes, ((0, 0), (0, 0), (0, _NP - _N)))
    anc = jnp.pad(anchors, ((0, _NP - _N), (0, 0))).T[:, None, :]
    keys, vals = _prep(planes, anc)
    cand = _sc_call(keys.reshape(-1), vals.reshape(-1))
    out = _nms(cand.reshape(_B, _CW))
    return out.reshape(_POST, 5, b).transpose(2, 0, 1)
